# baseline (speedup 1.0000x reference)
import jax
import jax.numpy as jnp
from jax import lax
from jax.experimental import pallas as pl
from jax.experimental.pallas import tpu as pltpu
from jax.experimental.pallas import tpu_sc as plsc

N = 10000
E = 320000
D = 200
H = 80
R = 8
T = 6

NC = 2
NS = 16
NW = NC * NS
EPW = E // NW
CH = 80
NCHUNK = EPW // CH
NPAD = 10240
ROWS_PT = NPAD // NS
LANES = 16


def _rel_transform_body(x_ref, wbig_ref, wself_ref, src_ref, typ_ref,
                        hrel_ref, xw_ref, gidx_ref):
    xb = x_ref[...]
    hrel_ref[...] = jnp.dot(xb, wbig_ref[...], preferred_element_type=jnp.float32)
    xw_ref[...] = jnp.dot(xb, wself_ref[...], preferred_element_type=jnp.float32)
    gidx_ref[...] = src_ref[...] * R + typ_ref[...]


def _edge_agg_body(hrel_hbm, gidx_hbm, dst_hbm, nrm_hbm, out_hbm,
                   gbuf, dbuf, nbuf, rows_v, zero_v, acc_sh,
                   sg0, sg1, sg2, sm0, sm1, sm2, ss0, ss1, ss2):
    c = lax.axis_index("c")
    s = lax.axis_index("s")
    wid = s * NC + c
    base = wid * EPW
    sem_g = (sg0, sg1, sg2)
    sem_m = (sm0, sm1, sm2)
    sem_s = (ss0, ss1, ss2)

    zf = jnp.zeros((LANES,), jnp.float32)

    def zrow(j, carry):
        for k in range(H // LANES):
            zero_v[j, pl.ds(k * LANES, LANES)] = zf
        return carry

    lax.fori_loop(0, CH, zrow, 0)
    row0 = s * ROWS_PT
    for k in range(ROWS_PT // CH):
        pltpu.sync_copy(zero_v.at[pl.ds(0, CH)],
                        acc_sh.at[pl.ds(row0 + k * CH, CH)])
    plsc.subcore_barrier()

    def meta_copies(j, ph):
        off = pl.multiple_of(base + j * CH, 8)
        return (
            pltpu.make_async_copy(gidx_hbm.at[pl.ds(off, CH)], gbuf.at[ph],
                                  sem_m[ph]),
            pltpu.make_async_copy(dst_hbm.at[pl.ds(off, CH)], dbuf.at[ph],
                                  sem_m[ph]),
            pltpu.make_async_copy(nrm_hbm.at[pl.ds(off, CH)],
                                  nbuf.at[pl.ds(ph * CH, CH)], sem_m[ph]),
        )

    def issue_meta(j, ph):
        for cp in meta_copies(j, ph):
            cp.start()

    def wait_meta(j, ph):
        for cp in meta_copies(j, ph):
            cp.wait()

    def gather_copy(ph):
        return pltpu.make_async_copy(hrel_hbm.at[gbuf.at[ph]],
                                     rows_v.at[ph], sem_g[ph])

    def scatter_copy(ph):
        return pltpu.make_async_copy(rows_v.at[ph], acc_sh.at[dbuf.at[ph]],
                                     sem_s[ph])

    def scale(ph):
        def edge8(e8, carry):
            for u in range(8):
                e = e8 * 8 + u
                spl = plsc.load_gather(
                    nbuf, [jnp.full((LANES,), ph * CH + e, jnp.int32)])
                for k in range(H // LANES):
                    sl = pl.ds(k * LANES, LANES)
                    rows_v[ph, e, sl] = rows_v[ph, e, sl] * spl
            return carry

        lax.fori_loop(0, CH // 8, edge8, 0)

    issue_meta(0, 0)
    wait_meta(0, 0)
    gather_copy(0).start()
    issue_meta(1, 1)

    def triple(t, carry):
        j0 = t * 3
        for ph in range(3):
            j = j0 + ph
            nph = (ph + 1) % 3
            pph = (ph + 2) % 3
            wait_meta(j + 1, nph)
            gather_copy(nph).start()

            @pl.when(j >= 1)
            def _():
                scatter_copy(pph).wait()

            issue_meta(j + 2, pph)
            gather_copy(ph).wait()
            scale(ph)
            scatter_copy(ph).start(add=True)
        return carry

    lax.fori_loop(0, (NCHUNK - 2) // 3, triple, 0)

    ph_a = (NCHUNK - 2) % 3
    ph_b = (NCHUNK - 1) % 3
    wait_meta(NCHUNK - 1, ph_b)
    gather_copy(ph_b).start()
    gather_copy(ph_a).wait()
    scale(ph_a)
    scatter_copy((NCHUNK - 3) % 3).wait()
    scatter_copy(ph_a).start(add=True)
    gather_copy(ph_b).wait()
    scale(ph_b)
    scatter_copy(ph_a).wait()
    scatter_copy(ph_b).start(add=True)
    scatter_copy(ph_b).wait()

    plsc.subcore_barrier()
    pltpu.sync_copy(acc_sh.at[pl.ds(row0, ROWS_PT)],
                    out_hbm.at[c, pl.ds(row0, ROWS_PT)])


def _finish_body(p_ref, xw_ref, b_ref, wout_ref, bout_ref, out_ref):
    h = p_ref[0] + p_ref[1] + xw_ref[...] + b_ref[...]
    h = jnp.maximum(h, 0.0)
    logits = jnp.dot(h, wout_ref[...], preferred_element_type=jnp.float32)
    logits = logits + bout_ref[...]
    m = jnp.max(logits, axis=1, keepdims=True)
    lse = jnp.log(jnp.sum(jnp.exp(logits - m), axis=1, keepdims=True)) + m
    out_ref[...] = logits - lse


def kernel(x, edge_index, edge_norm, edge_type, W_rel, W_self, b, W_out, b_out):
    src = edge_index[0].astype(jnp.int32)
    dst = edge_index[1].astype(jnp.int32)
    typ = edge_type.astype(jnp.int32)
    nrm = edge_norm.astype(jnp.float32)
    W_big = jnp.transpose(W_rel, (1, 0, 2)).reshape(D, R * H)

    BN = 1000
    NB = N // BN
    EB = E // NB
    src2 = src.reshape(NB, 1, EB)
    typ2 = typ.reshape(NB, 1, EB)
    hrel, xw, gidx2 = pl.pallas_call(
        _rel_transform_body,
        grid=(NB,),
        in_specs=[pl.BlockSpec((BN, D), lambda i: (i, 0)),
                  pl.BlockSpec((D, R * H), lambda i: (0, 0)),
                  pl.BlockSpec((D, H), lambda i: (0, 0)),
                  pl.BlockSpec((1, 1, EB), lambda i: (i, 0, 0)),
                  pl.BlockSpec((1, 1, EB), lambda i: (i, 0, 0))],
        out_specs=[pl.BlockSpec((BN, R * H), lambda i: (i, 0)),
                   pl.BlockSpec((BN, H), lambda i: (i, 0)),
                   pl.BlockSpec((1, 1, EB), lambda i: (i, 0, 0))],
        out_shape=[jax.ShapeDtypeStruct((N, R * H), jnp.float32),
                   jax.ShapeDtypeStruct((N, H), jnp.float32),
                   jax.ShapeDtypeStruct((NB, 1, EB), jnp.int32)],
    )(x, W_big, W_self, src2, typ2)
    hrel_flat = hrel.reshape(R * N, H)
    gidx = gidx2.reshape(E)

    mesh = plsc.VectorSubcoreMesh(core_axis_name="c", subcore_axis_name="s",
                                  num_cores=NC, num_subcores=NS)
    agg2 = pl.kernel(
        _edge_agg_body,
        out_type=jax.ShapeDtypeStruct((NC, NPAD, H), jnp.float32),
        mesh=mesh,
        compiler_params=pltpu.CompilerParams(use_tc_tiling_on_sc=False,
                                             needs_layout_passes=False),
        scratch_types=[
            pltpu.VMEM((3, CH), jnp.int32),
            pltpu.VMEM((3, CH), jnp.int32),
            pltpu.VMEM((3 * CH,), jnp.float32),
            pltpu.VMEM((3, CH, H), jnp.float32),
            pltpu.VMEM((CH, H), jnp.float32),
            pltpu.VMEM_SHARED((NPAD, H), jnp.float32),
            pltpu.SemaphoreType.DMA,
            pltpu.SemaphoreType.DMA,
            pltpu.SemaphoreType.DMA,
            pltpu.SemaphoreType.DMA,
            pltpu.SemaphoreType.DMA,
            pltpu.SemaphoreType.DMA,
            pltpu.SemaphoreType.DMA,
            pltpu.SemaphoreType.DMA,
            pltpu.SemaphoreType.DMA,
        ],
    )(hrel_flat, gidx, dst, nrm)

    out = pl.pallas_call(
        _finish_body,
        grid=(NB,),
        in_specs=[pl.BlockSpec((NC, BN, H), lambda i: (0, i, 0)),
                  pl.BlockSpec((BN, H), lambda i: (i, 0)),
                  pl.BlockSpec((1, H), lambda i: (0, 0)),
                  pl.BlockSpec((H, T), lambda i: (0, 0)),
                  pl.BlockSpec((1, T), lambda i: (0, 0))],
        out_specs=pl.BlockSpec((BN, T), lambda i: (i, 0)),
        out_shape=jax.ShapeDtypeStruct((N, T), jnp.float32),
    )(agg2, xw, b.reshape(1, H), W_out, b_out.reshape(1, T))
    return out


# speedup vs baseline: 1.0021x; 1.0021x over previous
import jax
import jax.numpy as jnp
from jax import lax
from jax.experimental import pallas as pl
from jax.experimental.pallas import tpu as pltpu
from jax.experimental.pallas import tpu_sc as plsc

N = 10000
E = 320000
D = 200
H = 80
R = 8
T = 6

NC = 2
NS = 16
NW = NC * NS
EPW = E // NW
CH = 80
NCHUNK = EPW // CH
NPAD = 10240
ROWS_PT = NPAD // NS
LANES = 16


def _rel_transform_body(x_ref, wbig_ref, wself_ref, src_ref, typ_ref,
                        hrel_ref, xw_ref, gidx_ref):
    xb = x_ref[...]
    hrel_ref[...] = jnp.dot(xb, wbig_ref[...], preferred_element_type=jnp.float32)
    xw_ref[...] = jnp.dot(xb, wself_ref[...], preferred_element_type=jnp.float32)
    gidx_ref[...] = src_ref[...] * R + typ_ref[...]


def _edge_agg_body(hrel_hbm, gidx_hbm, dst_hbm, nrm_hbm, out_hbm,
                   gbuf, dbuf, nbuf, rows_v, zero_v, acc_sh,
                   sg0, sg1, sg2, sm0, sm1, sm2, ss0, ss1, ss2):
    c = lax.axis_index("c")
    s = lax.axis_index("s")
    wid = s * NC + c
    base = wid * EPW
    sem_g = (sg0, sg1, sg2)
    sem_m = (sm0, sm1, sm2)
    sem_s = (ss0, ss1, ss2)

    zf = jnp.zeros((LANES,), jnp.float32)

    def zrow(j, carry):
        for k in range(H // LANES):
            zero_v[j, pl.ds(k * LANES, LANES)] = zf
        return carry

    lax.fori_loop(0, CH, zrow, 0)
    row0 = s * ROWS_PT
    for k in range(ROWS_PT // CH):
        pltpu.sync_copy(zero_v.at[pl.ds(0, CH)],
                        acc_sh.at[pl.ds(row0 + k * CH, CH)])
    plsc.subcore_barrier()

    def meta_copies(j, ph):
        off = pl.multiple_of(base + j * CH, 8)
        return (
            pltpu.make_async_copy(gidx_hbm.at[pl.ds(off, CH)], gbuf.at[ph],
                                  sem_m[ph]),
            pltpu.make_async_copy(dst_hbm.at[pl.ds(off, CH)], dbuf.at[ph],
                                  sem_m[ph]),
            pltpu.make_async_copy(nrm_hbm.at[pl.ds(off, CH)],
                                  nbuf.at[pl.ds(ph * CH, CH)], sem_m[ph]),
        )

    def issue_meta(j, ph):
        for cp in meta_copies(j, ph):
            cp.start()

    def wait_meta(j, ph):
        for cp in meta_copies(j, ph):
            cp.wait()

    def gather_copy(ph):
        return pltpu.make_async_copy(hrel_hbm.at[gbuf.at[ph]],
                                     rows_v.at[ph], sem_g[ph])

    def scatter_copy(ph):
        return pltpu.make_async_copy(rows_v.at[ph], acc_sh.at[dbuf.at[ph]],
                                     sem_s[ph])

    def scale(ph):
        def edge4(e4, carry):
            for u in range(4):
                e = e4 * 4 + u
                spl = plsc.load_gather(
                    nbuf, [jnp.full((LANES,), ph * CH, jnp.int32)
                           + jnp.full((LANES,), e, jnp.int32)])
                for k in range(H // LANES):
                    sl = pl.ds(k * LANES, LANES)
                    rows_v[ph, e, sl] = rows_v[ph, e, sl] * spl
            return carry

        lax.fori_loop(0, CH // 4, edge4, 0)

    issue_meta(0, 0)
    wait_meta(0, 0)
    gather_copy(0).start()
    issue_meta(1, 1)

    def triple(t, carry):
        j0 = t * 3
        for ph in range(3):
            j = j0 + ph
            nph = (ph + 1) % 3
            pph = (ph + 2) % 3
            wait_meta(j + 1, nph)
            gather_copy(nph).start()

            @pl.when(j >= 1)
            def _():
                scatter_copy(pph).wait()

            issue_meta(j + 2, pph)
            gather_copy(ph).wait()
            scale(ph)
            scatter_copy(ph).start(add=True)
        return carry

    lax.fori_loop(0, (NCHUNK - 2) // 3, triple, 0)

    ph_a = (NCHUNK - 2) % 3
    ph_b = (NCHUNK - 1) % 3
    wait_meta(NCHUNK - 1, ph_b)
    gather_copy(ph_b).start()
    gather_copy(ph_a).wait()
    scale(ph_a)
    scatter_copy((NCHUNK - 3) % 3).wait()
    scatter_copy(ph_a).start(add=True)
    gather_copy(ph_b).wait()
    scale(ph_b)
    scatter_copy(ph_a).wait()
    scatter_copy(ph_b).start(add=True)
    scatter_copy(ph_b).wait()

    plsc.subcore_barrier()
    pltpu.sync_copy(acc_sh.at[pl.ds(row0, ROWS_PT)],
                    out_hbm.at[c, pl.ds(row0, ROWS_PT)])


def _finish_body(p_ref, xw_ref, b_ref, wout_ref, bout_ref, out_ref):
    h = p_ref[0] + p_ref[1] + xw_ref[...] + b_ref[...]
    h = jnp.maximum(h, 0.0)
    logits = jnp.dot(h, wout_ref[...], preferred_element_type=jnp.float32)
    logits = logits + bout_ref[...]
    m = jnp.max(logits, axis=1, keepdims=True)
    lse = jnp.log(jnp.sum(jnp.exp(logits - m), axis=1, keepdims=True)) + m
    out_ref[...] = logits - lse


def kernel(x, edge_index, edge_norm, edge_type, W_rel, W_self, b, W_out, b_out):
    src = edge_index[0].astype(jnp.int32)
    dst = edge_index[1].astype(jnp.int32)
    typ = edge_type.astype(jnp.int32)
    nrm = edge_norm.astype(jnp.float32)
    W_big = jnp.transpose(W_rel, (1, 0, 2)).reshape(D, R * H)

    BN = 1000
    NB = N // BN
    EB = E // NB
    src2 = src.reshape(NB, 1, EB)
    typ2 = typ.reshape(NB, 1, EB)
    hrel, xw, gidx2 = pl.pallas_call(
        _rel_transform_body,
        grid=(NB,),
        in_specs=[pl.BlockSpec((BN, D), lambda i: (i, 0)),
                  pl.BlockSpec((D, R * H), lambda i: (0, 0)),
                  pl.BlockSpec((D, H), lambda i: (0, 0)),
                  pl.BlockSpec((1, 1, EB), lambda i: (i, 0, 0)),
                  pl.BlockSpec((1, 1, EB), lambda i: (i, 0, 0))],
        out_specs=[pl.BlockSpec((BN, R * H), lambda i: (i, 0)),
                   pl.BlockSpec((BN, H), lambda i: (i, 0)),
                   pl.BlockSpec((1, 1, EB), lambda i: (i, 0, 0))],
        out_shape=[jax.ShapeDtypeStruct((N, R * H), jnp.float32),
                   jax.ShapeDtypeStruct((N, H), jnp.float32),
                   jax.ShapeDtypeStruct((NB, 1, EB), jnp.int32)],
    )(x, W_big, W_self, src2, typ2)
    hrel_flat = hrel.reshape(R * N, H)
    gidx = gidx2.reshape(E)

    mesh = plsc.VectorSubcoreMesh(core_axis_name="c", subcore_axis_name="s",
                                  num_cores=NC, num_subcores=NS)
    agg2 = pl.kernel(
        _edge_agg_body,
        out_type=jax.ShapeDtypeStruct((NC, NPAD, H), jnp.float32),
        mesh=mesh,
        compiler_params=pltpu.CompilerParams(use_tc_tiling_on_sc=False,
                                             needs_layout_passes=False),
        scratch_types=[
            pltpu.VMEM((3, CH), jnp.int32),
            pltpu.VMEM((3, CH), jnp.int32),
            pltpu.VMEM((3 * CH,), jnp.float32),
            pltpu.VMEM((3, CH, H), jnp.float32),
            pltpu.VMEM((CH, H), jnp.float32),
            pltpu.VMEM_SHARED((NPAD, H), jnp.float32),
            pltpu.SemaphoreType.DMA,
            pltpu.SemaphoreType.DMA,
            pltpu.SemaphoreType.DMA,
            pltpu.SemaphoreType.DMA,
            pltpu.SemaphoreType.DMA,
            pltpu.SemaphoreType.DMA,
            pltpu.SemaphoreType.DMA,
            pltpu.SemaphoreType.DMA,
            pltpu.SemaphoreType.DMA,
        ],
    )(hrel_flat, gidx, dst, nrm)

    out = pl.pallas_call(
        _finish_body,
        grid=(NB,),
        in_specs=[pl.BlockSpec((NC, BN, H), lambda i: (0, i, 0)),
                  pl.BlockSpec((BN, H), lambda i: (i, 0)),
                  pl.BlockSpec((1, H), lambda i: (0, 0)),
                  pl.BlockSpec((H, T), lambda i: (0, 0)),
                  pl.BlockSpec((1, T), lambda i: (0, 0))],
        out_specs=pl.BlockSpec((BN, T), lambda i: (i, 0)),
        out_shape=jax.ShapeDtypeStruct((N, T), jnp.float32),
    )(agg2, xw, b.reshape(1, H), W_out, b_out.reshape(1, T))
    return out
